# baseline (speedup 1.0000x reference)
import jax
import jax.numpy as jnp
from jax.experimental import pallas as pl
from jax.experimental.pallas import tpu as pltpu


def _gcn_kernel(x_ref, a0_ref, a1_ref, w_ref, b_ref, out_ref, xw_ref):
    i = pl.program_id(0)

    @pl.when(i == 0)
    def _compute_xw():
        xw_ref[...] = jax.lax.dot(
            x_ref[...], w_ref[...], preferred_element_type=jnp.float32
        )

    hm = a0_ref.shape[0]
    out_ref[:hm, :] = jax.lax.dot(
        a0_ref[...], xw_ref[...], preferred_element_type=jnp.float32
    ) + b_ref[...]
    out_ref[hm:, :] = jax.lax.dot(
        a1_ref[...], xw_ref[...], preferred_element_type=jnp.float32
    ) + b_ref[...]


def _pick_block(n):
    for b in (400, 200, 100, 8, 4, 2, 1):
        if n % b == 0:
            return b
    return n


def kernel(input, adj, weight, bias):
    n, f_in = input.shape
    f_out = weight.shape[1]
    bm = _pick_block(n)
    hm = bm // 2
    bias2 = bias.reshape(1, f_out)
    grid = (n // bm,)
    return pl.pallas_call(
        _gcn_kernel,
        grid=grid,
        in_specs=[
            pl.BlockSpec((n, f_in), lambda i: (0, 0)),
            pl.BlockSpec((hm, n), lambda i: (2 * i, 0)),
            pl.BlockSpec((hm, n), lambda i: (2 * i + 1, 0)),
            pl.BlockSpec((f_in, f_out), lambda i: (0, 0)),
            pl.BlockSpec((1, f_out), lambda i: (0, 0)),
        ],
        out_specs=pl.BlockSpec((bm, f_out), lambda i: (i, 0)),
        out_shape=jax.ShapeDtypeStruct((n, f_out), jnp.float32),
        scratch_shapes=[pltpu.VMEM((n, f_out), jnp.float32)],
    )(input, adj, adj, weight, bias2)


# speedup vs baseline: 1.0177x; 1.0177x over previous
import jax
import jax.numpy as jnp
from jax.experimental import pallas as pl
from jax.experimental.pallas import tpu as pltpu


def _gcn_kernel(x_ref, adj_ref, w_ref, b_ref, out_ref, xw_ref):
    i = pl.program_id(0)

    @pl.when(i == 0)
    def _compute_xw():
        xw_ref[...] = jax.lax.dot(
            x_ref[...], w_ref[...], preferred_element_type=jnp.float32
        )

    acc = jax.lax.dot(adj_ref[...], xw_ref[...],
                      preferred_element_type=jnp.float32)
    out_ref[...] = acc + b_ref[...]


def _pick_block(n):
    for b in (400, 200, 100, 8, 4, 2, 1):
        if n % b == 0:
            return b
    return n


def kernel(input, adj, weight, bias):
    n, f_in = input.shape
    f_out = weight.shape[1]
    bm = _pick_block(n)
    bias2 = bias.reshape(1, f_out)
    grid = (n // bm,)
    return pl.pallas_call(
        _gcn_kernel,
        grid=grid,
        in_specs=[
            pl.BlockSpec((n, f_in), lambda i: (0, 0)),
            pl.BlockSpec((bm, n), lambda i: (i, 0)),
            pl.BlockSpec((f_in, f_out), lambda i: (0, 0)),
            pl.BlockSpec((1, f_out), lambda i: (0, 0)),
        ],
        out_specs=pl.BlockSpec((bm, f_out), lambda i: (i, 0)),
        out_shape=jax.ShapeDtypeStruct((n, f_out), jnp.float32),
        scratch_shapes=[pltpu.VMEM((n, f_out), jnp.float32)],
    )(input, adj, weight, bias2)
